# baseline (speedup 1.0000x reference)
import jax
import jax.numpy as jnp
from jax import lax
from jax.experimental import pallas as pl
from jax.experimental.pallas import tpu as pltpu
from jax.experimental.pallas import tpu_sc as plsc

N_NODES = 10000
N_EDGES = 320000
DIM = 128
HALF = 64
N_CLS = 10

NC = 2
NS = 16
IB = 640
JB = 2
NCHUNK = 16
TILE_IDX_ROWS = NCHUNK * JB
IDX_ROWS = NS * TILE_IDX_ROWS
E_PAD = IDX_ROWS * IB
ACC_ROWS = N_NODES + 8
ROWS_A = 640
ROWS_B = N_NODES - (NS - 1) * ROWS_A

BLK = 1000
GRID = N_NODES // BLK


def _segsum_body(x2, srcs2, dst2, out, gidx, didx, rows, acc, gsem, ssem):
    c = lax.axis_index("c")
    s = lax.axis_index("s")
    r0 = pl.multiple_of(s * ROWS_A, 8)
    xbase = pl.multiple_of(c * N_NODES + r0, 8)

    @pl.when(s < NS - 1)
    def _():
        pltpu.sync_copy(x2.at[pl.ds(xbase, ROWS_A)],
                        acc.at[pl.ds(r0, ROWS_A)])

    @pl.when(s == NS - 1)
    def _():
        pltpu.sync_copy(x2.at[pl.ds(xbase, ROWS_B)],
                        acc.at[pl.ds(r0, ROWS_B)])

    plsc.subcore_barrier()

    src_base = pl.multiple_of(c * IDX_ROWS + s * TILE_IDX_ROWS, 8)
    dst_base = pl.multiple_of(s * TILE_IDX_ROWS, 8)

    def chunk(i, carry):
        pltpu.sync_copy(srcs2.at[pl.ds(src_base + i * JB, JB)], gidx)
        pltpu.sync_copy(dst2.at[pl.ds(dst_base + i * JB, JB)], didx)
        gats = [pltpu.async_copy(x2.at[gidx.at[h]], rows.at[h], gsem)
                for h in range(JB)]
        scat = []
        for h in range(JB):
            gats[h].wait()
            scat.append(pltpu.async_copy(
                rows.at[h], acc.at[didx.at[h]], ssem, add=True))
        for cp in scat:
            cp.wait()
        return carry

    lax.fori_loop(0, NCHUNK, chunk, 0)
    plsc.subcore_barrier()

    @pl.when(s < NS - 1)
    def _():
        pltpu.sync_copy(acc.at[pl.ds(r0, ROWS_A)],
                        out.at[pl.ds(xbase, ROWS_A)])

    @pl.when(s == NS - 1)
    def _():
        pltpu.sync_copy(acc.at[pl.ds(r0, ROWS_B)],
                        out.at[pl.ds(xbase, ROWS_B)])


_segsum = pl.kernel(
    _segsum_body,
    out_type=jax.ShapeDtypeStruct((2 * N_NODES, HALF), jnp.float32),
    mesh=plsc.VectorSubcoreMesh(core_axis_name="c", subcore_axis_name="s",
                                num_cores=NC, num_subcores=NS),
    compiler_params=pltpu.CompilerParams(use_tc_tiling_on_sc=False),
    scratch_types=[
        pltpu.VMEM((JB, IB), jnp.int32),
        pltpu.VMEM((JB, IB), jnp.int32),
        pltpu.VMEM((JB, IB, HALF), jnp.float32),
        pltpu.VMEM_SHARED((ACC_ROWS, HALF), jnp.float32),
        pltpu.SemaphoreType.DMA,
        pltpu.SemaphoreType.DMA,
    ],
)


def _mlp_body(hin_ref, w1_ref, b1_ref, w2_ref, b2_ref,
              p_ref, sum_ref, ssq_ref):
    hin = jnp.concatenate([hin_ref[0], hin_ref[1]], axis=1)
    h = jnp.dot(hin, w1_ref[...], preferred_element_type=jnp.float32)
    h = jnp.maximum(h + b1_ref[...], 0.0)
    p = jnp.dot(h, w2_ref[...], preferred_element_type=jnp.float32)
    p = p + b2_ref[...]
    p_ref[...] = p

    @pl.when(pl.program_id(0) == 0)
    def _():
        sum_ref[...] = jnp.zeros_like(sum_ref)
        ssq_ref[...] = jnp.zeros_like(ssq_ref)

    sum_ref[...] += jnp.sum(p, axis=0, keepdims=True)
    ssq_ref[...] += jnp.sum(p * p, axis=0, keepdims=True)


def _mlp(parts, w1, b1, w2, b2):
    return pl.pallas_call(
        _mlp_body,
        grid=(GRID,),
        in_specs=[
            pl.BlockSpec((2, BLK, HALF), lambda i: (0, i, 0)),
            pl.BlockSpec((DIM, DIM), lambda i: (0, 0)),
            pl.BlockSpec((1, DIM), lambda i: (0, 0)),
            pl.BlockSpec((DIM, DIM), lambda i: (0, 0)),
            pl.BlockSpec((1, DIM), lambda i: (0, 0)),
        ],
        out_specs=[
            pl.BlockSpec((BLK, DIM), lambda i: (i, 0)),
            pl.BlockSpec((1, DIM), lambda i: (0, 0)),
            pl.BlockSpec((1, DIM), lambda i: (0, 0)),
        ],
        out_shape=[
            jax.ShapeDtypeStruct((N_NODES, DIM), jnp.float32),
            jax.ShapeDtypeStruct((1, DIM), jnp.float32),
            jax.ShapeDtypeStruct((1, DIM), jnp.float32),
        ],
    )(parts, w1, b1, w2, b2)


def _bn_coeffs(sum_ref, ssq_ref, g_ref, be_ref):
    mean = sum_ref[...] * (1.0 / N_NODES)
    var = ssq_ref[...] * (1.0 / N_NODES) - mean * mean
    scale = g_ref[...] * lax.rsqrt(var + 1e-5)
    shift = be_ref[...] - mean * scale
    return scale, shift


def _bnrelu_body(p_ref, sum_ref, ssq_ref, g_ref, be_ref, out_ref):
    scale, shift = _bn_coeffs(sum_ref, ssq_ref, g_ref, be_ref)
    h = jnp.maximum(p_ref[...] * scale + shift, 0.0)
    out_ref[0] = h[:, :HALF]
    out_ref[1] = h[:, HALF:]


def _bnrelu(p, s, ssq, g, be):
    return pl.pallas_call(
        _bnrelu_body,
        grid=(GRID,),
        in_specs=[
            pl.BlockSpec((BLK, DIM), lambda i: (i, 0)),
            pl.BlockSpec((1, DIM), lambda i: (0, 0)),
            pl.BlockSpec((1, DIM), lambda i: (0, 0)),
            pl.BlockSpec((1, DIM), lambda i: (0, 0)),
            pl.BlockSpec((1, DIM), lambda i: (0, 0)),
        ],
        out_specs=pl.BlockSpec((2, BLK, HALF), lambda i: (0, i, 0)),
        out_shape=jax.ShapeDtypeStruct((2, N_NODES, HALF), jnp.float32),
    )(p, s, ssq, g, be)


def _head_body(p_ref, sum_ref, ssq_ref, g_ref, be_ref,
               wc1_ref, bc1_ref, wc2_ref, bc2_ref,
               latent_ref, class_ref):
    scale, shift = _bn_coeffs(sum_ref, ssq_ref, g_ref, be_ref)
    latent = jnp.maximum(p_ref[...] * scale + shift, 0.0)
    latent_ref[...] = latent
    cmid = jnp.dot(latent, wc1_ref[...], preferred_element_type=jnp.float32)
    cmid = jnp.maximum(cmid + bc1_ref[...], 0.0)
    cls = jnp.dot(cmid, wc2_ref[...], preferred_element_type=jnp.float32)
    class_ref[...] = cls + bc2_ref[...]


def _head(p, s, ssq, g, be, wc1, bc1, wc2, bc2):
    return pl.pallas_call(
        _head_body,
        grid=(GRID,),
        in_specs=[
            pl.BlockSpec((BLK, DIM), lambda i: (i, 0)),
            pl.BlockSpec((1, DIM), lambda i: (0, 0)),
            pl.BlockSpec((1, DIM), lambda i: (0, 0)),
            pl.BlockSpec((1, DIM), lambda i: (0, 0)),
            pl.BlockSpec((1, DIM), lambda i: (0, 0)),
            pl.BlockSpec((DIM, DIM), lambda i: (0, 0)),
            pl.BlockSpec((1, DIM), lambda i: (0, 0)),
            pl.BlockSpec((DIM, N_CLS), lambda i: (0, 0)),
            pl.BlockSpec((1, N_CLS), lambda i: (0, 0)),
        ],
        out_specs=[
            pl.BlockSpec((BLK, DIM), lambda i: (i, 0)),
            pl.BlockSpec((BLK, N_CLS), lambda i: (i, 0)),
        ],
        out_shape=[
            jax.ShapeDtypeStruct((N_NODES, DIM), jnp.float32),
            jax.ShapeDtypeStruct((N_NODES, N_CLS), jnp.float32),
        ],
    )(p, s, ssq, g, be, wc1, bc1, wc2, bc2)


@jax.jit
def kernel(x, edge_index, W11, b11, W12, b12, W21, b21, W22, b22,
           g1, be1, g2, be2, Wc1, bc1, Wc2, bc2):
    src = edge_index[0]
    dst = edge_index[1]
    npad = E_PAD - N_EDGES
    src_p = jnp.concatenate([src, jnp.zeros((npad,), jnp.int32)])
    dst_p = jnp.concatenate([dst, jnp.full((npad,), N_NODES, jnp.int32)])
    srcs2 = jnp.concatenate([src_p, src_p + N_NODES]).reshape(-1, IB)
    dst2 = dst_p.reshape(-1, IB)

    row = lambda v: v.reshape(1, -1)
    b11r, b12r, b21r, b22r = row(b11), row(b12), row(b21), row(b22)
    g1r, be1r, g2r, be2r = row(g1), row(be1), row(g2), row(be2)
    bc1r, bc2r = row(bc1), row(bc2)

    x2 = jnp.concatenate([x[:, :HALF], x[:, HALF:]], axis=0)
    hin1 = _segsum(x2, srcs2, dst2).reshape(2, N_NODES, HALF)
    p1, s1, q1 = _mlp(hin1, W11, b11r, W12, b12r)
    h1 = _bnrelu(p1, s1, q1, g1r, be1r)
    hin2 = _segsum(h1.reshape(2 * N_NODES, HALF), srcs2, dst2)
    p2, s2, q2 = _mlp(hin2.reshape(2, N_NODES, HALF), W21, b21r, W22, b22r)
    latent, class_out = _head(p2, s2, q2, g2r, be2r, Wc1, bc1r, Wc2, bc2r)
    return (latent, class_out)


# speedup vs baseline: 1.6447x; 1.6447x over previous
import jax
import jax.numpy as jnp
from jax import lax
from jax.experimental import pallas as pl
from jax.experimental.pallas import tpu as pltpu
from jax.experimental.pallas import tpu_sc as plsc

N_NODES = 10000
N_EDGES = 320000
DIM = 128
QW = 32
NQ = DIM // QW
N_CLS = 10

NC = 2
NS = 16
IB = 1024
JB = 2
NCHUNK = 10
TILE_IDX_ROWS = NCHUNK * JB
IDX_ROWS = NS * TILE_IDX_ROWS
E_PAD = IDX_ROWS * IB
ACC_ROWS = N_NODES + 8
ROWS_A = 640
ROWS_B = N_NODES - (NS - 1) * ROWS_A

BLK = 1000
GRID = N_NODES // BLK


def _segsum_body(xq, srcs2, dst2, out, gidx, didx, rows, tq, acc, gsem, ssem):
    c = lax.axis_index("c")
    s = lax.axis_index("s")
    r0 = pl.multiple_of(s * ROWS_A, 8)
    base = pl.multiple_of(s * TILE_IDX_ROWS, 8)

    for p in range(2):
        qbase = pl.multiple_of((c * 2 + p) * N_NODES + r0, 8)

        @pl.when(s < NS - 1)
        def _():
            pltpu.sync_copy(xq.at[pl.ds(qbase, ROWS_A)],
                            tq.at[pl.ds(r0, ROWS_A)])
            pltpu.sync_copy(xq.at[pl.ds(qbase, ROWS_A)],
                            acc.at[pl.ds(r0, ROWS_A)])

        @pl.when(s == NS - 1)
        def _():
            pltpu.sync_copy(xq.at[pl.ds(qbase, ROWS_B)],
                            tq.at[pl.ds(r0, ROWS_B)])
            pltpu.sync_copy(xq.at[pl.ds(qbase, ROWS_B)],
                            acc.at[pl.ds(r0, ROWS_B)])

        plsc.subcore_barrier()

        def chunk(i, carry):
            pltpu.sync_copy(srcs2.at[pl.ds(base + i * JB, JB)], gidx)
            pltpu.sync_copy(dst2.at[pl.ds(base + i * JB, JB)], didx)
            scat = []
            for h in range(JB):
                g = pltpu.async_copy(tq.at[gidx.at[h]], rows.at[h], gsem)
                g.wait()
                scat.append(pltpu.async_copy(
                    rows.at[h], acc.at[didx.at[h]], ssem, add=True))
            for cp in scat:
                cp.wait()
            return carry

        lax.fori_loop(0, NCHUNK, chunk, 0)
        plsc.subcore_barrier()

        @pl.when(s < NS - 1)
        def _():
            pltpu.sync_copy(acc.at[pl.ds(r0, ROWS_A)],
                            out.at[pl.ds(qbase, ROWS_A)])

        @pl.when(s == NS - 1)
        def _():
            pltpu.sync_copy(acc.at[pl.ds(r0, ROWS_B)],
                            out.at[pl.ds(qbase, ROWS_B)])

        plsc.subcore_barrier()


_segsum = pl.kernel(
    _segsum_body,
    out_type=jax.ShapeDtypeStruct((NQ * N_NODES, QW), jnp.float32),
    mesh=plsc.VectorSubcoreMesh(core_axis_name="c", subcore_axis_name="s",
                                num_cores=NC, num_subcores=NS),
    compiler_params=pltpu.CompilerParams(use_tc_tiling_on_sc=False),
    scratch_types=[
        pltpu.VMEM((JB, IB), jnp.int32),
        pltpu.VMEM((JB, IB), jnp.int32),
        pltpu.VMEM((JB, IB, QW), jnp.float32),
        pltpu.VMEM_SHARED((ACC_ROWS, QW), jnp.float32),
        pltpu.VMEM_SHARED((ACC_ROWS, QW), jnp.float32),
        pltpu.SemaphoreType.DMA,
        pltpu.SemaphoreType.DMA,
    ],
)


def _mlp_body(hin_ref, w1_ref, b1_ref, w2_ref, b2_ref,
              p_ref, sum_ref, ssq_ref):
    hin = jnp.concatenate([hin_ref[k] for k in range(NQ)], axis=1)
    h = jnp.dot(hin, w1_ref[...], preferred_element_type=jnp.float32)
    h = jnp.maximum(h + b1_ref[...], 0.0)
    p = jnp.dot(h, w2_ref[...], preferred_element_type=jnp.float32)
    p = p + b2_ref[...]
    p_ref[...] = p

    @pl.when(pl.program_id(0) == 0)
    def _():
        sum_ref[...] = jnp.zeros_like(sum_ref)
        ssq_ref[...] = jnp.zeros_like(ssq_ref)

    sum_ref[...] += jnp.sum(p, axis=0, keepdims=True)
    ssq_ref[...] += jnp.sum(p * p, axis=0, keepdims=True)


def _mlp(parts, w1, b1, w2, b2):
    return pl.pallas_call(
        _mlp_body,
        grid=(GRID,),
        in_specs=[
            pl.BlockSpec((NQ, BLK, QW), lambda i: (0, i, 0)),
            pl.BlockSpec((DIM, DIM), lambda i: (0, 0)),
            pl.BlockSpec((1, DIM), lambda i: (0, 0)),
            pl.BlockSpec((DIM, DIM), lambda i: (0, 0)),
            pl.BlockSpec((1, DIM), lambda i: (0, 0)),
        ],
        out_specs=[
            pl.BlockSpec((BLK, DIM), lambda i: (i, 0)),
            pl.BlockSpec((1, DIM), lambda i: (0, 0)),
            pl.BlockSpec((1, DIM), lambda i: (0, 0)),
        ],
        out_shape=[
            jax.ShapeDtypeStruct((N_NODES, DIM), jnp.float32),
            jax.ShapeDtypeStruct((1, DIM), jnp.float32),
            jax.ShapeDtypeStruct((1, DIM), jnp.float32),
        ],
    )(parts, w1, b1, w2, b2)


def _bn_coeffs(sum_ref, ssq_ref, g_ref, be_ref):
    mean = sum_ref[...] * (1.0 / N_NODES)
    var = ssq_ref[...] * (1.0 / N_NODES) - mean * mean
    scale = g_ref[...] * lax.rsqrt(var + 1e-5)
    shift = be_ref[...] - mean * scale
    return scale, shift


def _bnrelu_body(p_ref, sum_ref, ssq_ref, g_ref, be_ref, out_ref):
    scale, shift = _bn_coeffs(sum_ref, ssq_ref, g_ref, be_ref)
    h = jnp.maximum(p_ref[...] * scale + shift, 0.0)
    for k in range(NQ):
        out_ref[k] = h[:, k * QW:(k + 1) * QW]


def _bnrelu(p, s, ssq, g, be):
    return pl.pallas_call(
        _bnrelu_body,
        grid=(GRID,),
        in_specs=[
            pl.BlockSpec((BLK, DIM), lambda i: (i, 0)),
            pl.BlockSpec((1, DIM), lambda i: (0, 0)),
            pl.BlockSpec((1, DIM), lambda i: (0, 0)),
            pl.BlockSpec((1, DIM), lambda i: (0, 0)),
            pl.BlockSpec((1, DIM), lambda i: (0, 0)),
        ],
        out_specs=pl.BlockSpec((NQ, BLK, QW), lambda i: (0, i, 0)),
        out_shape=jax.ShapeDtypeStruct((NQ, N_NODES, QW), jnp.float32),
    )(p, s, ssq, g, be)


def _head_body(p_ref, sum_ref, ssq_ref, g_ref, be_ref,
               wc1_ref, bc1_ref, wc2_ref, bc2_ref,
               latent_ref, class_ref):
    scale, shift = _bn_coeffs(sum_ref, ssq_ref, g_ref, be_ref)
    latent = jnp.maximum(p_ref[...] * scale + shift, 0.0)
    latent_ref[...] = latent
    cmid = jnp.dot(latent, wc1_ref[...], preferred_element_type=jnp.float32)
    cmid = jnp.maximum(cmid + bc1_ref[...], 0.0)
    cls = jnp.dot(cmid, wc2_ref[...], preferred_element_type=jnp.float32)
    class_ref[...] = cls + bc2_ref[...]


def _head(p, s, ssq, g, be, wc1, bc1, wc2, bc2):
    return pl.pallas_call(
        _head_body,
        grid=(GRID,),
        in_specs=[
            pl.BlockSpec((BLK, DIM), lambda i: (i, 0)),
            pl.BlockSpec((1, DIM), lambda i: (0, 0)),
            pl.BlockSpec((1, DIM), lambda i: (0, 0)),
            pl.BlockSpec((1, DIM), lambda i: (0, 0)),
            pl.BlockSpec((1, DIM), lambda i: (0, 0)),
            pl.BlockSpec((DIM, DIM), lambda i: (0, 0)),
            pl.BlockSpec((1, DIM), lambda i: (0, 0)),
            pl.BlockSpec((DIM, N_CLS), lambda i: (0, 0)),
            pl.BlockSpec((1, N_CLS), lambda i: (0, 0)),
        ],
        out_specs=[
            pl.BlockSpec((BLK, DIM), lambda i: (i, 0)),
            pl.BlockSpec((BLK, N_CLS), lambda i: (i, 0)),
        ],
        out_shape=[
            jax.ShapeDtypeStruct((N_NODES, DIM), jnp.float32),
            jax.ShapeDtypeStruct((N_NODES, N_CLS), jnp.float32),
        ],
    )(p, s, ssq, g, be, wc1, bc1, wc2, bc2)


@jax.jit
def kernel(x, edge_index, W11, b11, W12, b12, W21, b21, W22, b22,
           g1, be1, g2, be2, Wc1, bc1, Wc2, bc2):
    src = edge_index[0]
    dst = edge_index[1]
    npad = E_PAD - N_EDGES
    srcs2 = jnp.concatenate(
        [src, jnp.zeros((npad,), jnp.int32)]).reshape(-1, IB)
    dst2 = jnp.concatenate(
        [dst, jnp.full((npad,), N_NODES, jnp.int32)]).reshape(-1, IB)

    row = lambda v: v.reshape(1, -1)
    b11r, b12r, b21r, b22r = row(b11), row(b12), row(b21), row(b22)
    g1r, be1r, g2r, be2r = row(g1), row(be1), row(g2), row(be2)
    bc1r, bc2r = row(bc1), row(bc2)

    xq = x.reshape(N_NODES, NQ, QW).transpose(1, 0, 2).reshape(-1, QW)
    hin1 = _segsum(xq, srcs2, dst2).reshape(NQ, N_NODES, QW)
    p1, s1, q1 = _mlp(hin1, W11, b11r, W12, b12r)
    h1 = _bnrelu(p1, s1, q1, g1r, be1r)
    hin2 = _segsum(h1.reshape(NQ * N_NODES, QW), srcs2, dst2)
    p2, s2, q2 = _mlp(hin2.reshape(NQ, N_NODES, QW), W21, b21r, W22, b22r)
    latent, class_out = _head(p2, s2, q2, g2r, be2r, Wc1, bc1r, Wc2, bc2r)
    return (latent, class_out)


# speedup vs baseline: 1.7431x; 1.0598x over previous
import jax
import jax.numpy as jnp
from jax import lax
from jax.experimental import pallas as pl
from jax.experimental.pallas import tpu as pltpu
from jax.experimental.pallas import tpu_sc as plsc

N_NODES = 10000
N_EDGES = 320000
DIM = 128
QW = 32
NQ = DIM // QW
N_CLS = 10

NC = 2
NS = 16
IB = 1000
JB = 2
NCHUNK = 10
TILE_IDX_ROWS = NCHUNK * JB
IDX_ROWS = NS * TILE_IDX_ROWS
ACC_ROWS = N_NODES + 8
ROWS_A = 640
ROWS_B = N_NODES - (NS - 1) * ROWS_A

BLK = 1000
GRID = N_NODES // BLK


def _segsum_body(xq, srcs2, dst2, out, gidx, didx, rows, tq, acc, gsem, ssem):
    c = lax.axis_index("c")
    s = lax.axis_index("s")
    r0 = pl.multiple_of(s * ROWS_A, 8)
    base = pl.multiple_of(s * TILE_IDX_ROWS, 8)

    for p in range(2):
        qbase = pl.multiple_of((c * 2 + p) * N_NODES + r0, 8)

        @pl.when(s < NS - 1)
        def _():
            pltpu.sync_copy(xq.at[pl.ds(qbase, ROWS_A)],
                            tq.at[pl.ds(r0, ROWS_A)])
            pltpu.sync_copy(xq.at[pl.ds(qbase, ROWS_A)],
                            acc.at[pl.ds(r0, ROWS_A)])

        @pl.when(s == NS - 1)
        def _():
            pltpu.sync_copy(xq.at[pl.ds(qbase, ROWS_B)],
                            tq.at[pl.ds(r0, ROWS_B)])
            pltpu.sync_copy(xq.at[pl.ds(qbase, ROWS_B)],
                            acc.at[pl.ds(r0, ROWS_B)])

        plsc.subcore_barrier()

        def chunk(i, carry):
            pltpu.sync_copy(srcs2.at[pl.ds(base + i * JB, JB)], gidx)
            pltpu.sync_copy(dst2.at[pl.ds(base + i * JB, JB)], didx)
            scat = []
            for h in range(JB):
                g = pltpu.async_copy(tq.at[gidx.at[h]], rows.at[h], gsem)
                g.wait()
                scat.append(pltpu.async_copy(
                    rows.at[h], acc.at[didx.at[h]], ssem, add=True))
            for cp in scat:
                cp.wait()
            return carry

        lax.fori_loop(0, NCHUNK, chunk, 0)
        plsc.subcore_barrier()

        @pl.when(s < NS - 1)
        def _():
            pltpu.sync_copy(acc.at[pl.ds(r0, ROWS_A)],
                            out.at[pl.ds(qbase, ROWS_A)])

        @pl.when(s == NS - 1)
        def _():
            pltpu.sync_copy(acc.at[pl.ds(r0, ROWS_B)],
                            out.at[pl.ds(qbase, ROWS_B)])

        plsc.subcore_barrier()


_segsum = pl.kernel(
    _segsum_body,
    out_type=jax.ShapeDtypeStruct((NQ * N_NODES, QW), jnp.float32),
    mesh=plsc.VectorSubcoreMesh(core_axis_name="c", subcore_axis_name="s",
                                num_cores=NC, num_subcores=NS),
    compiler_params=pltpu.CompilerParams(use_tc_tiling_on_sc=False),
    scratch_types=[
        pltpu.VMEM((JB, IB), jnp.int32),
        pltpu.VMEM((JB, IB), jnp.int32),
        pltpu.VMEM((JB, IB, QW), jnp.float32),
        pltpu.VMEM_SHARED((ACC_ROWS, QW), jnp.float32),
        pltpu.VMEM_SHARED((ACC_ROWS, QW), jnp.float32),
        pltpu.SemaphoreType.DMA,
        pltpu.SemaphoreType.DMA,
    ],
)


def _mlp_body(hin_ref, w1_ref, b1_ref, w2_ref, b2_ref,
              p_ref, sum_ref, ssq_ref):
    hin = jnp.concatenate([hin_ref[k] for k in range(NQ)], axis=1)
    h = jnp.dot(hin, w1_ref[...], preferred_element_type=jnp.float32)
    h = jnp.maximum(h + b1_ref[...], 0.0)
    p = jnp.dot(h, w2_ref[...], preferred_element_type=jnp.float32)
    p = p + b2_ref[...]
    p_ref[...] = p

    @pl.when(pl.program_id(0) == 0)
    def _():
        sum_ref[...] = jnp.zeros_like(sum_ref)
        ssq_ref[...] = jnp.zeros_like(ssq_ref)

    sum_ref[...] += jnp.sum(p, axis=0, keepdims=True)
    ssq_ref[...] += jnp.sum(p * p, axis=0, keepdims=True)


def _mlp(parts, w1, b1, w2, b2):
    return pl.pallas_call(
        _mlp_body,
        grid=(GRID,),
        in_specs=[
            pl.BlockSpec((NQ, BLK, QW), lambda i: (0, i, 0)),
            pl.BlockSpec((DIM, DIM), lambda i: (0, 0)),
            pl.BlockSpec((1, DIM), lambda i: (0, 0)),
            pl.BlockSpec((DIM, DIM), lambda i: (0, 0)),
            pl.BlockSpec((1, DIM), lambda i: (0, 0)),
        ],
        out_specs=[
            pl.BlockSpec((BLK, DIM), lambda i: (i, 0)),
            pl.BlockSpec((1, DIM), lambda i: (0, 0)),
            pl.BlockSpec((1, DIM), lambda i: (0, 0)),
        ],
        out_shape=[
            jax.ShapeDtypeStruct((N_NODES, DIM), jnp.float32),
            jax.ShapeDtypeStruct((1, DIM), jnp.float32),
            jax.ShapeDtypeStruct((1, DIM), jnp.float32),
        ],
    )(parts, w1, b1, w2, b2)


def _bn_coeffs(sum_ref, ssq_ref, g_ref, be_ref):
    mean = sum_ref[...] * (1.0 / N_NODES)
    var = ssq_ref[...] * (1.0 / N_NODES) - mean * mean
    scale = g_ref[...] * lax.rsqrt(var + 1e-5)
    shift = be_ref[...] - mean * scale
    return scale, shift


def _bnrelu_body(p_ref, sum_ref, ssq_ref, g_ref, be_ref, out_ref):
    scale, shift = _bn_coeffs(sum_ref, ssq_ref, g_ref, be_ref)
    h = jnp.maximum(p_ref[...] * scale + shift, 0.0)
    for k in range(NQ):
        out_ref[k] = h[:, k * QW:(k + 1) * QW]


def _bnrelu(p, s, ssq, g, be):
    return pl.pallas_call(
        _bnrelu_body,
        grid=(GRID,),
        in_specs=[
            pl.BlockSpec((BLK, DIM), lambda i: (i, 0)),
            pl.BlockSpec((1, DIM), lambda i: (0, 0)),
            pl.BlockSpec((1, DIM), lambda i: (0, 0)),
            pl.BlockSpec((1, DIM), lambda i: (0, 0)),
            pl.BlockSpec((1, DIM), lambda i: (0, 0)),
        ],
        out_specs=pl.BlockSpec((NQ, BLK, QW), lambda i: (0, i, 0)),
        out_shape=jax.ShapeDtypeStruct((NQ, N_NODES, QW), jnp.float32),
    )(p, s, ssq, g, be)


def _head_body(p_ref, sum_ref, ssq_ref, g_ref, be_ref,
               wc1_ref, bc1_ref, wc2_ref, bc2_ref,
               latent_ref, class_ref):
    scale, shift = _bn_coeffs(sum_ref, ssq_ref, g_ref, be_ref)
    latent = jnp.maximum(p_ref[...] * scale + shift, 0.0)
    latent_ref[...] = latent
    cmid = jnp.dot(latent, wc1_ref[...], preferred_element_type=jnp.float32)
    cmid = jnp.maximum(cmid + bc1_ref[...], 0.0)
    cls = jnp.dot(cmid, wc2_ref[...], preferred_element_type=jnp.float32)
    class_ref[...] = cls + bc2_ref[...]


def _head(p, s, ssq, g, be, wc1, bc1, wc2, bc2):
    return pl.pallas_call(
        _head_body,
        grid=(GRID,),
        in_specs=[
            pl.BlockSpec((BLK, DIM), lambda i: (i, 0)),
            pl.BlockSpec((1, DIM), lambda i: (0, 0)),
            pl.BlockSpec((1, DIM), lambda i: (0, 0)),
            pl.BlockSpec((1, DIM), lambda i: (0, 0)),
            pl.BlockSpec((1, DIM), lambda i: (0, 0)),
            pl.BlockSpec((DIM, DIM), lambda i: (0, 0)),
            pl.BlockSpec((1, DIM), lambda i: (0, 0)),
            pl.BlockSpec((DIM, N_CLS), lambda i: (0, 0)),
            pl.BlockSpec((1, N_CLS), lambda i: (0, 0)),
        ],
        out_specs=[
            pl.BlockSpec((BLK, DIM), lambda i: (i, 0)),
            pl.BlockSpec((BLK, N_CLS), lambda i: (i, 0)),
        ],
        out_shape=[
            jax.ShapeDtypeStruct((N_NODES, DIM), jnp.float32),
            jax.ShapeDtypeStruct((N_NODES, N_CLS), jnp.float32),
        ],
    )(p, s, ssq, g, be, wc1, bc1, wc2, bc2)


@jax.jit
def kernel(x, edge_index, W11, b11, W12, b12, W21, b21, W22, b22,
           g1, be1, g2, be2, Wc1, bc1, Wc2, bc2):
    src = edge_index[0]
    dst = edge_index[1]
    srcs2 = src.reshape(-1, IB)
    dst2 = dst.reshape(-1, IB)

    row = lambda v: v.reshape(1, -1)
    b11r, b12r, b21r, b22r = row(b11), row(b12), row(b21), row(b22)
    g1r, be1r, g2r, be2r = row(g1), row(be1), row(g2), row(be2)
    bc1r, bc2r = row(bc1), row(bc2)

    xq = x.reshape(N_NODES, NQ, QW).transpose(1, 0, 2).reshape(-1, QW)
    hin1 = _segsum(xq, srcs2, dst2).reshape(NQ, N_NODES, QW)
    p1, s1, q1 = _mlp(hin1, W11, b11r, W12, b12r)
    h1 = _bnrelu(p1, s1, q1, g1r, be1r)
    hin2 = _segsum(h1.reshape(NQ * N_NODES, QW), srcs2, dst2)
    p2, s2, q2 = _mlp(hin2.reshape(NQ, N_NODES, QW), W21, b21r, W22, b22r)
    latent, class_out = _head(p2, s2, q2, g2r, be2r, Wc1, bc1r, Wc2, bc2r)
    return (latent, class_out)


# speedup vs baseline: 1.9947x; 1.1443x over previous
import jax
import jax.numpy as jnp
from jax import lax
from jax.experimental import pallas as pl
from jax.experimental.pallas import tpu as pltpu
from jax.experimental.pallas import tpu_sc as plsc

N_NODES = 10000
N_EDGES = 320000
DIM = 128
QW = 32
N_CLS = 10

NC = 2
NS = 16
IB = 1000
JB = 2
NCHUNK = 10
TILE_IDX_ROWS = NCHUNK * JB
ACC_ROWS = N_NODES + 8
ROWS_A = 640
ROWS_B = N_NODES - (NS - 1) * ROWS_A

BLK = 1000
GRID = N_NODES // BLK


def _segsum_body(xs, srcs2, dst2, out, gidx, didx, rows, tq, acc, gsem, ssem):
    c = lax.axis_index("c")
    s = lax.axis_index("s")
    r0 = pl.multiple_of(s * ROWS_A, 8)
    base = pl.multiple_of(s * TILE_IDX_ROWS, 8)

    for p in range(2):
        col = pl.multiple_of((c * 2 + p) * QW, 8)

        @pl.when(s < NS - 1)
        def _():
            pltpu.sync_copy(xs.at[pl.ds(r0, ROWS_A), pl.ds(col, QW)],
                            tq.at[pl.ds(r0, ROWS_A)])
            pltpu.sync_copy(xs.at[pl.ds(r0, ROWS_A), pl.ds(col, QW)],
                            acc.at[pl.ds(r0, ROWS_A)])

        @pl.when(s == NS - 1)
        def _():
            pltpu.sync_copy(xs.at[pl.ds(r0, ROWS_B), pl.ds(col, QW)],
                            tq.at[pl.ds(r0, ROWS_B)])
            pltpu.sync_copy(xs.at[pl.ds(r0, ROWS_B), pl.ds(col, QW)],
                            acc.at[pl.ds(r0, ROWS_B)])

        plsc.subcore_barrier()

        def chunk(i, carry):
            pltpu.sync_copy(srcs2.at[pl.ds(base + i * JB, JB)], gidx)
            pltpu.sync_copy(dst2.at[pl.ds(base + i * JB, JB)], didx)
            scat = []
            for h in range(JB):
                g = pltpu.async_copy(tq.at[gidx.at[h]], rows.at[h], gsem)
                g.wait()
                scat.append(pltpu.async_copy(
                    rows.at[h], acc.at[didx.at[h]], ssem, add=True))
            for cp in scat:
                cp.wait()
            return carry

        lax.fori_loop(0, NCHUNK, chunk, 0)
        plsc.subcore_barrier()

        @pl.when(s < NS - 1)
        def _():
            pltpu.sync_copy(acc.at[pl.ds(r0, ROWS_A)],
                            out.at[pl.ds(r0, ROWS_A), pl.ds(col, QW)])

        @pl.when(s == NS - 1)
        def _():
            pltpu.sync_copy(acc.at[pl.ds(r0, ROWS_B)],
                            out.at[pl.ds(r0, ROWS_B), pl.ds(col, QW)])

        plsc.subcore_barrier()


_segsum = pl.kernel(
    _segsum_body,
    out_type=jax.ShapeDtypeStruct((N_NODES, DIM), jnp.float32),
    mesh=plsc.VectorSubcoreMesh(core_axis_name="c", subcore_axis_name="s",
                                num_cores=NC, num_subcores=NS),
    compiler_params=pltpu.CompilerParams(use_tc_tiling_on_sc=False),
    scratch_types=[
        pltpu.VMEM((JB, IB), jnp.int32),
        pltpu.VMEM((JB, IB), jnp.int32),
        pltpu.VMEM((JB, IB, QW), jnp.float32),
        pltpu.VMEM_SHARED((ACC_ROWS, QW), jnp.float32),
        pltpu.VMEM_SHARED((ACC_ROWS, QW), jnp.float32),
        pltpu.SemaphoreType.DMA,
        pltpu.SemaphoreType.DMA,
    ],
)


def _mlp_body(hin_ref, w1_ref, b1_ref, w2_ref, b2_ref,
              p_ref, sum_ref, ssq_ref):
    h = jnp.dot(hin_ref[...], w1_ref[...], preferred_element_type=jnp.float32)
    h = jnp.maximum(h + b1_ref[...], 0.0)
    p = jnp.dot(h, w2_ref[...], preferred_element_type=jnp.float32)
    p = p + b2_ref[...]
    p_ref[...] = p

    @pl.when(pl.program_id(0) == 0)
    def _():
        sum_ref[...] = jnp.zeros_like(sum_ref)
        ssq_ref[...] = jnp.zeros_like(ssq_ref)

    sum_ref[...] += jnp.sum(p, axis=0, keepdims=True)
    ssq_ref[...] += jnp.sum(p * p, axis=0, keepdims=True)


def _mlp(hin, w1, b1, w2, b2):
    return pl.pallas_call(
        _mlp_body,
        grid=(GRID,),
        in_specs=[
            pl.BlockSpec((BLK, DIM), lambda i: (i, 0)),
            pl.BlockSpec((DIM, DIM), lambda i: (0, 0)),
            pl.BlockSpec((1, DIM), lambda i: (0, 0)),
            pl.BlockSpec((DIM, DIM), lambda i: (0, 0)),
            pl.BlockSpec((1, DIM), lambda i: (0, 0)),
        ],
        out_specs=[
            pl.BlockSpec((BLK, DIM), lambda i: (i, 0)),
            pl.BlockSpec((1, DIM), lambda i: (0, 0)),
            pl.BlockSpec((1, DIM), lambda i: (0, 0)),
        ],
        out_shape=[
            jax.ShapeDtypeStruct((N_NODES, DIM), jnp.float32),
            jax.ShapeDtypeStruct((1, DIM), jnp.float32),
            jax.ShapeDtypeStruct((1, DIM), jnp.float32),
        ],
    )(hin, w1, b1, w2, b2)


def _bn_coeffs(sum_ref, ssq_ref, g_ref, be_ref):
    mean = sum_ref[...] * (1.0 / N_NODES)
    var = ssq_ref[...] * (1.0 / N_NODES) - mean * mean
    scale = g_ref[...] * lax.rsqrt(var + 1e-5)
    shift = be_ref[...] - mean * scale
    return scale, shift


def _bnrelu_body(p_ref, sum_ref, ssq_ref, g_ref, be_ref, out_ref):
    scale, shift = _bn_coeffs(sum_ref, ssq_ref, g_ref, be_ref)
    out_ref[...] = jnp.maximum(p_ref[...] * scale + shift, 0.0)


def _bnrelu(p, s, ssq, g, be):
    return pl.pallas_call(
        _bnrelu_body,
        grid=(GRID,),
        in_specs=[
            pl.BlockSpec((BLK, DIM), lambda i: (i, 0)),
            pl.BlockSpec((1, DIM), lambda i: (0, 0)),
            pl.BlockSpec((1, DIM), lambda i: (0, 0)),
            pl.BlockSpec((1, DIM), lambda i: (0, 0)),
            pl.BlockSpec((1, DIM), lambda i: (0, 0)),
        ],
        out_specs=pl.BlockSpec((BLK, DIM), lambda i: (i, 0)),
        out_shape=jax.ShapeDtypeStruct((N_NODES, DIM), jnp.float32),
    )(p, s, ssq, g, be)


def _head_body(p_ref, sum_ref, ssq_ref, g_ref, be_ref,
               wc1_ref, bc1_ref, wc2_ref, bc2_ref,
               latent_ref, class_ref):
    scale, shift = _bn_coeffs(sum_ref, ssq_ref, g_ref, be_ref)
    latent = jnp.maximum(p_ref[...] * scale + shift, 0.0)
    latent_ref[...] = latent
    cmid = jnp.dot(latent, wc1_ref[...], preferred_element_type=jnp.float32)
    cmid = jnp.maximum(cmid + bc1_ref[...], 0.0)
    cls = jnp.dot(cmid, wc2_ref[...], preferred_element_type=jnp.float32)
    class_ref[...] = cls + bc2_ref[...]


def _head(p, s, ssq, g, be, wc1, bc1, wc2, bc2):
    return pl.pallas_call(
        _head_body,
        grid=(GRID,),
        in_specs=[
            pl.BlockSpec((BLK, DIM), lambda i: (i, 0)),
            pl.BlockSpec((1, DIM), lambda i: (0, 0)),
            pl.BlockSpec((1, DIM), lambda i: (0, 0)),
            pl.BlockSpec((1, DIM), lambda i: (0, 0)),
            pl.BlockSpec((1, DIM), lambda i: (0, 0)),
            pl.BlockSpec((DIM, DIM), lambda i: (0, 0)),
            pl.BlockSpec((1, DIM), lambda i: (0, 0)),
            pl.BlockSpec((DIM, N_CLS), lambda i: (0, 0)),
            pl.BlockSpec((1, N_CLS), lambda i: (0, 0)),
        ],
        out_specs=[
            pl.BlockSpec((BLK, DIM), lambda i: (i, 0)),
            pl.BlockSpec((BLK, N_CLS), lambda i: (i, 0)),
        ],
        out_shape=[
            jax.ShapeDtypeStruct((N_NODES, DIM), jnp.float32),
            jax.ShapeDtypeStruct((N_NODES, N_CLS), jnp.float32),
        ],
    )(p, s, ssq, g, be, wc1, bc1, wc2, bc2)


@jax.jit
def kernel(x, edge_index, W11, b11, W12, b12, W21, b21, W22, b22,
           g1, be1, g2, be2, Wc1, bc1, Wc2, bc2):
    src = edge_index[0]
    dst = edge_index[1]
    srcs2 = src.reshape(-1, IB)
    dst2 = dst.reshape(-1, IB)

    row = lambda v: v.reshape(1, -1)
    b11r, b12r, b21r, b22r = row(b11), row(b12), row(b21), row(b22)
    g1r, be1r, g2r, be2r = row(g1), row(be1), row(g2), row(be2)
    bc1r, bc2r = row(bc1), row(bc2)

    hin1 = _segsum(x, srcs2, dst2)
    p1, s1, q1 = _mlp(hin1, W11, b11r, W12, b12r)
    h1 = _bnrelu(p1, s1, q1, g1r, be1r)
    hin2 = _segsum(h1, srcs2, dst2)
    p2, s2, q2 = _mlp(hin2, W21, b21r, W22, b22r)
    latent, class_out = _head(p2, s2, q2, g2r, be2r, Wc1, bc1r, Wc2, bc2r)
    return (latent, class_out)
